# baseline (speedup 1.0000x reference)
import jax
import jax.numpy as jnp
from jax import lax
from jax.experimental import pallas as pl
from jax.experimental.pallas import tpu as pltpu
from jax.experimental.pallas import tpu_sc as plsc

_ROWS = 64
_N = 32768
_K = 256
_L = 16
_NCH = _N // _L
_ROWS_PER_W = 2
_PREFILTER = 2.25
_HI = 8.0
_NEG_INF = float("-inf")


def _sc_body(scores_hbm, out_hbm, x_v, out_v, pos_v, hist_v, totals_v, cnt_v,
             sc_smem, out_sem):
    c = lax.axis_index("c")
    s = lax.axis_index("s")
    wid = s * 2 + c

    iota = lax.iota(jnp.int32, _L)
    ones_i = jnp.ones((_L,), jnp.int32)
    zeros_i = jnp.zeros((_L,), jnp.int32)
    bankoff = iota * 256
    candoff = iota * _NCH
    one_f = jnp.ones((_L,), jnp.float32)
    zero_f = jnp.zeros((_L,), jnp.float32)

    def key_of(x):
        i = plsc.bitcast(x, jnp.int32)
        return i ^ (jnp.right_shift(i, 31) & jnp.int32(0x7FFFFFFF))

    def zero_hist():
        def zb(i, _):
            for u in range(8):
                hist_v[pl.ds((i * 8 + u) * _L, _L)] = zeros_i
            return 0
        lax.fori_loop(0, 4096 // _L // 8, zb, 0)

    def extract(v, j):
        return jnp.sum(jnp.where(iota == j, v, 0))

    def suffix(v):
        return jnp.flip(jnp.cumsum(jnp.flip(v)))

    def bank_reduce_and_select(r):
        def tb(j, _):
            acc = hist_v[pl.ds(j * _L, _L)]
            hist_v[pl.ds(j * _L, _L)] = zeros_i
            for lane in range(1, _L):
                acc = acc + hist_v[pl.ds(lane * 256 + j * _L, _L)]
                hist_v[pl.ds(lane * 256 + j * _L, _L)] = zeros_i
            totals_v[pl.ds(j * _L, _L)] = acc
            return 0
        lax.fori_loop(0, 256 // _L, tb, 0)
        cs = zeros_i
        for l in range(_L):
            cs = cs + plsc.load_gather(totals_v, [iota * _L + l])
        sfx = suffix(cs)
        jc = jnp.sum((sfx >= r).astype(jnp.int32)) - 1
        base = extract(sfx, jc) - extract(cs, jc)
        w = totals_v[pl.ds(jc * _L, _L)]
        tail = suffix(w)
        lsel = jnp.sum(((base + tail) >= r).astype(jnp.int32)) - 1
        bsel = jc * _L + lsel
        cnt_above = base + extract(tail, lsel) - extract(w, lsel)
        return bsel, r - cnt_above

    def collect(thresh_vec):
        hi_vec = jnp.full((_L,), _HI, jnp.float32)

        @plsc.parallel_loop(0, _NCH, 1, unroll=8, carry=(zeros_i, zeros_i))
        def final(i, carry):
            off, c8 = carry
            sl = pl.ds(i * _L, _L)
            x = x_v[sl]
            out_v[sl] = zero_f
            m = x >= thresh_vec
            pos = iota + i * _L
            plsc.store_scatter(pos_v, [candoff + off], pos, mask=m)
            c8 = c8 + (x >= hi_vec).astype(jnp.int32)
            return off + m.astype(jnp.int32), c8
        return final

    def do_row(row, prev_out_row):
        pltpu.sync_copy(scores_hbm.at[row], x_v)
        if prev_out_row is not None:
            pltpu.make_async_copy(out_v, out_hbm.at[prev_out_row],
                                  out_sem).wait()

        off, c8 = collect(jnp.full((_L,), _PREFILTER, jnp.float32))
        cnt_v[pl.ds(0, _L)] = off
        total = jnp.sum(off)
        cnt8 = jnp.sum(c8)
        fell_back = total < _K

        @pl.when(fell_back)
        def _():
            off2, _c8 = collect(jnp.full((_L,), _NEG_INF, jnp.float32))
            cnt_v[pl.ds(0, _L)] = off2

        cnts = cnt_v[pl.ds(0, _L)]
        maxc = jnp.max(cnts)
        need_l0 = fell_back | (cnt8 >= _K)

        def level(shift, prefix, r, first, cnts=cnts, maxc=maxc):

            def lb(i, _, shift=shift, prefix=prefix, first=first):
                p = plsc.load_gather(pos_v, [candoff + i])
                valid = cnts > i
                x = plsc.load_gather(x_v, [p], mask=valid)
                k = key_of(x)
                if first:
                    ok = valid
                    b = jnp.right_shift(k, 24) + 128
                else:
                    ok = valid & (jnp.right_shift(k, shift + 8) == prefix)
                    b = jnp.right_shift(k, shift) & 0xFF
                plsc.addupdate_scatter(hist_v, [bankoff + b], ones_i, mask=ok)
                return 0
            lax.fori_loop(0, maxc, lb, 0)
            return bank_reduce_and_select(r)

        @pl.when(need_l0)
        def _():
            bsel, r0 = level(24, jnp.int32(0), jnp.int32(_K), True)
            sc_smem[0] = bsel - 128
            sc_smem[1] = r0

        @pl.when(jnp.logical_not(need_l0))
        def _():
            sc_smem[0] = jnp.int32(64)
            sc_smem[1] = jnp.int32(_K) - cnt8

        prefix = sc_smem[0]
        r_l = sc_smem[1]
        for shift in (16, 8, 0):
            bsel, r_l = level(shift, prefix, r_l, False)
            prefix = (prefix << 8) | bsel

        tkey = jnp.broadcast_to(prefix, (_L,))

        def fb(i, eqc):
            p = plsc.load_gather(pos_v, [candoff + i])
            valid = cnts > i
            x = plsc.load_gather(x_v, [p], mask=valid)
            k = key_of(x)
            gt = valid & (k > tkey)
            plsc.store_scatter(out_v, [p], one_f, mask=gt)
            return eqc + (valid & (k == tkey)).astype(jnp.int32)
        eq = jnp.sum(lax.fori_loop(0, maxc, fb, zeros_i))

        @pl.when(eq == r_l)
        def _():
            def tb2(i, _):
                p = plsc.load_gather(pos_v, [candoff + i])
                valid = cnts > i
                x = plsc.load_gather(x_v, [p], mask=valid)
                tie = valid & (key_of(x) == tkey)
                plsc.store_scatter(out_v, [p], one_f, mask=tie)
                return 0
            lax.fori_loop(0, maxc, tb2, 0)

        @pl.when(eq != r_l)
        def _():
            def ob(i, run):
                sl = pl.ds(i * _L, _L)
                k = key_of(x_v[sl])
                gt = k > tkey
                tie = k == tkey
                incl = jnp.cumsum(tie.astype(jnp.int32))
                mark = tie & ((run + incl) <= r_l)
                out_v[sl] = jnp.where(gt | mark, one_f, zero_f)
                return run + jnp.sum(tie.astype(jnp.int32))
            lax.fori_loop(0, _NCH, ob, jnp.int32(0))

        pltpu.async_copy(out_v, out_hbm.at[row], out_sem)

    zero_hist()

    prev = None
    for j in range(_ROWS_PER_W):
        row = wid * _ROWS_PER_W + j
        do_row(row, prev)
        prev = row
    pltpu.make_async_copy(out_v, out_hbm.at[prev], out_sem).wait()


def kernel(scores):
    f = pl.kernel(
        _sc_body,
        out_type=jax.ShapeDtypeStruct((_ROWS, _N), jnp.float32),
        mesh=plsc.VectorSubcoreMesh(
            core_axis_name="c", subcore_axis_name="s",
            num_cores=2, num_subcores=16,
        ),
        scratch_types=[
            pltpu.VMEM((_N,), jnp.float32),
            pltpu.VMEM((_N,), jnp.float32),
            pltpu.VMEM((_N,), jnp.int32),
            pltpu.VMEM((4096,), jnp.int32),
            pltpu.VMEM((256,), jnp.int32),
            pltpu.VMEM((_L,), jnp.int32),
            pltpu.SMEM((4,), jnp.int32),
            pltpu.SemaphoreType.DMA,
        ],
        compiler_params=pltpu.CompilerParams(needs_layout_passes=False),
    )
    return f(scores)


# speedup vs baseline: 1.0423x; 1.0423x over previous
import jax
import jax.numpy as jnp
from jax import lax
from jax.experimental import pallas as pl
from jax.experimental.pallas import tpu as pltpu
from jax.experimental.pallas import tpu_sc as plsc

_ROWS = 64
_N = 32768
_K = 256
_L = 16
_NCH = _N // _L
_ROWS_PER_W = 2
_PREFILTER = 2.25
_HI = 8.0
_NEG_INF = float("-inf")


def _sc_body(scores_hbm, out_hbm, x_v, out_v, pos_v, hist_v, totals_v, cnt_v,
             sc_smem, out_sem):
    c = lax.axis_index("c")
    s = lax.axis_index("s")
    wid = s * 2 + c

    iota = lax.iota(jnp.int32, _L)
    ones_i = jnp.ones((_L,), jnp.int32)
    zeros_i = jnp.zeros((_L,), jnp.int32)
    bankoff = iota * 256
    candoff = iota * _NCH
    one_f = jnp.ones((_L,), jnp.float32)
    zero_f = jnp.zeros((_L,), jnp.float32)

    def key_of(x):
        i = plsc.bitcast(x, jnp.int32)
        return i ^ (jnp.right_shift(i, 31) & jnp.int32(0x7FFFFFFF))

    def zero_hist():
        def zb(i, _):
            for u in range(8):
                hist_v[pl.ds((i * 8 + u) * _L, _L)] = zeros_i
            return 0
        lax.fori_loop(0, 4096 // _L // 8, zb, 0)

    def extract(v, j):
        return jnp.sum(jnp.where(iota == j, v, 0))

    def suffix(v):
        return jnp.flip(jnp.cumsum(jnp.flip(v)))

    def bank_reduce_and_select(r):
        def tb(j, _):
            acc = hist_v[pl.ds(j * _L, _L)]
            hist_v[pl.ds(j * _L, _L)] = zeros_i
            for lane in range(1, _L):
                acc = acc + hist_v[pl.ds(lane * 256 + j * _L, _L)]
                hist_v[pl.ds(lane * 256 + j * _L, _L)] = zeros_i
            totals_v[pl.ds(j * _L, _L)] = acc
            return 0
        lax.fori_loop(0, 256 // _L, tb, 0)
        cs = zeros_i
        for l in range(_L):
            cs = cs + plsc.load_gather(totals_v, [iota * _L + l])
        sfx = suffix(cs)
        jc = jnp.sum((sfx >= r).astype(jnp.int32)) - 1
        base = extract(sfx, jc) - extract(cs, jc)
        w = totals_v[pl.ds(jc * _L, _L)]
        tail = suffix(w)
        lsel = jnp.sum(((base + tail) >= r).astype(jnp.int32)) - 1
        bsel = jc * _L + lsel
        cnt_above = base + extract(tail, lsel) - extract(w, lsel)
        return bsel, r - cnt_above

    def collect(thresh_vec):
        hi_vec = jnp.full((_L,), _HI, jnp.float32)

        @plsc.parallel_loop(0, _NCH, 1, unroll=8, carry=(zeros_i, zeros_i))
        def final(i, carry):
            off, c8 = carry
            x = x_v[pl.ds(i * _L, _L)]
            m = x >= thresh_vec
            pos = iota + i * _L
            plsc.store_scatter(pos_v, [candoff + off], pos, mask=m)
            c8 = c8 + (x >= hi_vec).astype(jnp.int32)
            return off + m.astype(jnp.int32), c8
        return final

    def do_row(row, prev_out_row):
        pltpu.sync_copy(scores_hbm.at[row], x_v)
        if prev_out_row is not None:
            pltpu.make_async_copy(out_v, out_hbm.at[prev_out_row],
                                  out_sem).wait()
            pcnts = cnt_v[pl.ds(0, _L)]
            pmaxc = jnp.max(pcnts)

            def zb(i, _):
                p = plsc.load_gather(pos_v, [candoff + i])
                valid = pcnts > i
                plsc.store_scatter(out_v, [p], zero_f, mask=valid)
                return 0
            lax.fori_loop(0, pmaxc, zb, 0)

        off, c8 = collect(jnp.full((_L,), _PREFILTER, jnp.float32))
        cnt_v[pl.ds(0, _L)] = off
        total = jnp.sum(off)
        cnt8 = jnp.sum(c8)
        fell_back = total < _K

        @pl.when(fell_back)
        def _():
            off2, _c8 = collect(jnp.full((_L,), _NEG_INF, jnp.float32))
            cnt_v[pl.ds(0, _L)] = off2

        cnts = cnt_v[pl.ds(0, _L)]
        maxc = jnp.max(cnts)
        need_l0 = fell_back | (cnt8 >= _K)

        def level(shift, prefix, r, first, cnts=cnts, maxc=maxc):

            def lb(i, _, shift=shift, prefix=prefix, first=first):
                p = plsc.load_gather(pos_v, [candoff + i])
                valid = cnts > i
                x = plsc.load_gather(x_v, [p], mask=valid)
                k = key_of(x)
                if first:
                    ok = valid
                    b = jnp.right_shift(k, 24) + 128
                else:
                    ok = valid & (jnp.right_shift(k, shift + 8) == prefix)
                    b = jnp.right_shift(k, shift) & 0xFF
                plsc.addupdate_scatter(hist_v, [bankoff + b], ones_i, mask=ok)
                return 0
            lax.fori_loop(0, maxc, lb, 0)
            return bank_reduce_and_select(r)

        @pl.when(need_l0)
        def _():
            bsel, r0 = level(24, jnp.int32(0), jnp.int32(_K), True)
            sc_smem[0] = bsel - 128
            sc_smem[1] = r0

        @pl.when(jnp.logical_not(need_l0))
        def _():
            sc_smem[0] = jnp.int32(64)
            sc_smem[1] = jnp.int32(_K) - cnt8

        prefix = sc_smem[0]
        r_l = sc_smem[1]
        for shift in (16, 8, 0):
            bsel, r_l = level(shift, prefix, r_l, False)
            prefix = (prefix << 8) | bsel

        tkey = jnp.broadcast_to(prefix, (_L,))

        def fb(i, eqc):
            p = plsc.load_gather(pos_v, [candoff + i])
            valid = cnts > i
            x = plsc.load_gather(x_v, [p], mask=valid)
            k = key_of(x)
            win = valid & (k >= tkey)
            plsc.store_scatter(out_v, [p], one_f, mask=win)
            return eqc + (valid & (k == tkey)).astype(jnp.int32)
        eq = jnp.sum(lax.fori_loop(0, maxc, fb, zeros_i))

        @pl.when(eq != r_l)
        def _():
            def ob(i, run):
                sl = pl.ds(i * _L, _L)
                k = key_of(x_v[sl])
                gt = k > tkey
                tie = k == tkey
                incl = jnp.cumsum(tie.astype(jnp.int32))
                mark = tie & ((run + incl) <= r_l)
                out_v[sl] = jnp.where(gt | mark, one_f, zero_f)
                return run + jnp.sum(tie.astype(jnp.int32))
            lax.fori_loop(0, _NCH, ob, jnp.int32(0))

        pltpu.async_copy(out_v, out_hbm.at[row], out_sem)

    zero_hist()

    @plsc.parallel_loop(0, _NCH, 1, unroll=8)
    def _zo(i):
        out_v[pl.ds(i * _L, _L)] = zero_f

    prev = None
    for j in range(_ROWS_PER_W):
        row = wid * _ROWS_PER_W + j
        do_row(row, prev)
        prev = row
    pltpu.make_async_copy(out_v, out_hbm.at[prev], out_sem).wait()


def kernel(scores):
    f = pl.kernel(
        _sc_body,
        out_type=jax.ShapeDtypeStruct((_ROWS, _N), jnp.float32),
        mesh=plsc.VectorSubcoreMesh(
            core_axis_name="c", subcore_axis_name="s",
            num_cores=2, num_subcores=16,
        ),
        scratch_types=[
            pltpu.VMEM((_N,), jnp.float32),
            pltpu.VMEM((_N,), jnp.float32),
            pltpu.VMEM((_N,), jnp.int32),
            pltpu.VMEM((4096,), jnp.int32),
            pltpu.VMEM((256,), jnp.int32),
            pltpu.VMEM((_L,), jnp.int32),
            pltpu.SMEM((4,), jnp.int32),
            pltpu.SemaphoreType.DMA,
        ],
        compiler_params=pltpu.CompilerParams(needs_layout_passes=False),
    )
    return f(scores)
